# baseline (speedup 1.0000x reference)
import functools

import jax
import jax.numpy as jnp
from jax import lax
from jax.experimental import pallas as pl
from jax.experimental.pallas import tpu as pltpu
from jax.experimental.pallas import tpu_sc as plsc

_VOCAB = 1000000
_HIDDEN = 64
_BATCH = 4096
_SEQ = 200

_TOT = _BATCH * _SEQ
_CH = 128
_NC = 2
_NS = 16
_NW = _NC * _NS
_CPW = _TOT // (_CH * _NW)

_mesh = plsc.VectorSubcoreMesh(core_axis_name="c", subcore_axis_name="s")


@functools.partial(
    pl.kernel,
    out_type=jax.ShapeDtypeStruct((_TOT, _HIDDEN), jnp.float32),
    mesh=_mesh,
    compiler_params=pltpu.CompilerParams(use_tc_tiling_on_sc=False),
    scratch_types=[
        pltpu.VMEM((_CPW, _CH), jnp.int32),
        pltpu.VMEM((_CH, _HIDDEN), jnp.float32),
        pltpu.SemaphoreType.DMA,
    ],
)
def _embed_gather(ids_hbm, table_hbm, out_hbm, idx_v, buf, sem):
    wid = lax.axis_index("s") * _NC + lax.axis_index("c")
    crow = wid * _CPW
    pltpu.sync_copy(ids_hbm.at[pl.ds(crow, _CPW)], idx_v)

    def step(j, carry):
        pltpu.async_copy(table_hbm.at[idx_v.at[j]], buf, sem).wait()
        pltpu.sync_copy(buf, out_hbm.at[pl.ds((crow + j) * _CH, _CH)])
        return carry

    lax.fori_loop(0, _CPW, step, 0)


def kernel(input_ids, token_embeddings):
    ids = input_ids.reshape(_TOT // _CH, _CH)
    out = _embed_gather(ids, token_embeddings)
    return out.reshape(_BATCH, _SEQ, _HIDDEN)


# speedup vs baseline: 1.1154x; 1.1154x over previous
import functools

import jax
import jax.numpy as jnp
from jax import lax
from jax.experimental import pallas as pl
from jax.experimental.pallas import tpu as pltpu
from jax.experimental.pallas import tpu_sc as plsc

_VOCAB = 1000000
_HIDDEN = 64
_BATCH = 4096
_SEQ = 200

_TOT = _BATCH * _SEQ
_CH = 128
_NC = 2
_NS = 16
_NW = _NC * _NS
_CPW = _TOT // (_CH * _NW)
_K = 4
_ROWS = _K * _CH
_NSUP = _CPW // _K

_mesh = plsc.VectorSubcoreMesh(core_axis_name="c", subcore_axis_name="s")


@functools.partial(
    pl.kernel,
    out_type=jax.ShapeDtypeStruct((_TOT, _HIDDEN), jnp.float32),
    mesh=_mesh,
    compiler_params=pltpu.CompilerParams(use_tc_tiling_on_sc=False),
    scratch_types=[
        pltpu.VMEM((_CPW, _CH), jnp.int32),
        pltpu.VMEM((_ROWS, _HIDDEN), jnp.float32),
        pltpu.VMEM((_ROWS, _HIDDEN), jnp.float32),
        pltpu.SemaphoreType.DMA,
        pltpu.SemaphoreType.DMA,
        pltpu.SemaphoreType.DMA,
        pltpu.SemaphoreType.DMA,
    ],
)
def _embed_gather(ids_hbm, table_hbm, out_hbm, idx_v, buf_a, buf_b,
                  gsem_a, gsem_b, psem_a, psem_b):
    wid = lax.axis_index("s") * _NC + lax.axis_index("c")
    crow = wid * _CPW
    pltpu.sync_copy(ids_hbm.at[pl.ds(crow, _CPW)], idx_v)

    def fire_gathers(s, buf, gsem):
        for b in range(_K):
            pltpu.make_async_copy(
                table_hbm.at[idx_v.at[s * _K + b]],
                buf.at[pl.ds(b * _CH, _CH)],
                gsem,
            ).start()

    def drain_gathers(buf, gsem):
        pltpu.make_async_copy(table_hbm.at[pl.ds(0, _ROWS)], buf, gsem).wait()

    def put(s, buf, psem):
        return pltpu.make_async_copy(
            buf, out_hbm.at[pl.ds((crow + s * _K) * _CH, _ROWS)], psem
        )

    fire_gathers(0, buf_a, gsem_a)

    def step(t, carry):
        s0 = 2 * t
        s1 = s0 + 1

        @pl.when(t > 0)
        def _():
            put(s1 - 2, buf_b, psem_b).wait()

        fire_gathers(s1, buf_b, gsem_b)
        drain_gathers(buf_a, gsem_a)
        put(s0, buf_a, psem_a).start()
        put(s0, buf_a, psem_a).wait()

        @pl.when(t < _NSUP // 2 - 1)
        def _():
            fire_gathers(s0 + 2, buf_a, gsem_a)

        drain_gathers(buf_b, gsem_b)
        put(s1, buf_b, psem_b).start()
        return carry

    lax.fori_loop(0, _NSUP // 2, step, 0)
    put(_NSUP - 1, buf_b, psem_b).wait()


def kernel(input_ids, token_embeddings):
    ids = input_ids.reshape(_TOT // _CH, _CH)
    out = _embed_gather(ids, token_embeddings)
    return out.reshape(_BATCH, _SEQ, _HIDDEN)
